# baseline (speedup 1.0000x reference)
import jax
import jax.numpy as jnp
from jax.experimental import pallas as pl


def _body(feats_ref, w_ref, out_ref):
    r = jax.lax.dot_general(
        w_ref[...], feats_ref[...], (((1,), (0,)), ((), ())),
        preferred_element_type=jnp.float32)
    s0 = r[:, 0:1]
    s1x = r[:, 1:2]
    s1y = r[:, 2:3]
    s2 = r[:, 3:4]
    var_sum = s2 + (s1x * s1x + s1y * s1y) * (s0 - 2.0)
    out_ref[...] = jnp.sum(var_sum, axis=0, keepdims=True) * (0.5 / r.shape[0])


def kernel(distribution, weights, spot_dist):
    del spot_dist
    x = distribution[:, 0:1]
    y = distribution[:, 1:2]
    feats = jnp.concatenate(
        [jnp.ones_like(x), x, y, x * x + y * y], axis=1)
    out = pl.pallas_call(
        _body,
        out_shape=jax.ShapeDtypeStruct((1, 1), jnp.float32),
    )(feats, weights)
    return out[0, 0]
